# Initial kernel scaffold; baseline (speedup 1.0000x reference)
import jax
import jax.numpy as jnp
from jax.experimental import pallas as pl


def kernel(x, linear_w, linear_bias, embed_table, W1, b1, W2, b2, W3, b3, Wout, bout):
    raise NotImplementedError("write your pallas kernel here")


# speedup vs baseline: 14.3688x; 14.3688x over previous
import functools

import jax
import jax.numpy as jnp
import numpy as np
from jax import lax
from jax.experimental import pallas as pl
from jax.experimental.pallas import tpu as pltpu
from jax.experimental.pallas import tpu_sc as plsc

B = 16384
F = 26
D = 128
FIELD = 4000
V = F * FIELD
EMB_OUT = F * D
B26 = B * F

NC = 2
NS = 16
NW = NC * NS
PER_W = B26 // NW
CH = 128
NCH = PER_W // CH
BW = B // NW


def _sc_emb_body(idx_hbm, tab_hbm, emb_out,
                 idx_v, bufA, bufB, semA, semB, semWA, semWB):
    wid = lax.axis_index("s") * NC + lax.axis_index("c")
    base = wid * PER_W
    pltpu.sync_copy(idx_hbm.at[wid], idx_v)

    def body(c, carry):
        c0 = c * 2
        c1 = c0 + 1

        @pl.when(c > 0)
        def _drain_prev_writes():
            pltpu.make_async_copy(bufA, emb_out.at[pl.ds(0, CH)], semWA).wait()
            pltpu.make_async_copy(bufB, emb_out.at[pl.ds(0, CH)], semWB).wait()

        gA = pltpu.async_copy(tab_hbm.at[idx_v.at[c0]], bufA, semA)
        gB = pltpu.async_copy(tab_hbm.at[idx_v.at[c1]], bufB, semB)
        gA.wait()
        pltpu.async_copy(bufA, emb_out.at[pl.ds(base + c0 * CH, CH)], semWA)
        gB.wait()
        pltpu.async_copy(bufB, emb_out.at[pl.ds(base + c1 * CH, CH)], semWB)
        return carry

    lax.fori_loop(0, NCH // 2, body, 0)
    pltpu.make_async_copy(bufA, emb_out.at[pl.ds(0, CH)], semWA).wait()
    pltpu.make_async_copy(bufB, emb_out.at[pl.ds(0, CH)], semWB).wait()


def _sc_wide_body(idx_hbm, lw_hbm, lin_out, idx_v, lw_v, acc_v, sem):
    wid = lax.axis_index("s") * NC + lax.axis_index("c")
    base = wid * BW
    cp_lw = pltpu.async_copy(lw_hbm, lw_v, sem)
    pltpu.sync_copy(idx_hbm.at[wid], idx_v)
    cp_lw.wait()

    def body(j, carry):
        p = j * 16
        acc = jnp.zeros((16,), jnp.float32)
        for f in range(F):
            ivec = idx_v[f, pl.ds(p, 16)]
            acc = acc + plsc.load_gather(lw_v, [ivec])
        acc_v[pl.ds(p, 16)] = acc
        return carry

    lax.fori_loop(0, BW // 16, body, 0)
    pltpu.sync_copy(acc_v, lin_out.at[pl.ds(base, BW)])


@functools.cache
def _get_sc_kernels():
    mesh = plsc.VectorSubcoreMesh(
        core_axis_name="c", subcore_axis_name="s",
        num_cores=NC, num_subcores=NS)
    emb_k = pl.kernel(
        _sc_emb_body,
        out_type=jax.ShapeDtypeStruct((B26, D), jnp.float32),
        mesh=mesh,
        scratch_types=[
            pltpu.VMEM((NCH, CH), jnp.int32),
            pltpu.VMEM((CH, D), jnp.float32),
            pltpu.VMEM((CH, D), jnp.float32),
            pltpu.SemaphoreType.DMA,
            pltpu.SemaphoreType.DMA,
            pltpu.SemaphoreType.DMA,
            pltpu.SemaphoreType.DMA,
        ],
    )
    wide_k = pl.kernel(
        _sc_wide_body,
        out_type=jax.ShapeDtypeStruct((B,), jnp.float32),
        mesh=mesh,
        scratch_types=[
            pltpu.VMEM((F, BW), jnp.int32),
            pltpu.VMEM((V,), jnp.float32),
            pltpu.VMEM((BW,), jnp.float32),
            pltpu.SemaphoreType.DMA,
        ],
        compiler_params=pltpu.CompilerParams(needs_layout_passes=False),
    )
    return emb_k, wide_k


BB = 512


def _mlp_body(emb_ref, lin_ref, w1_ref, b1_ref, w2_ref, b2_ref, w3_ref,
              b3_ref, wout_ref, cbias_ref, out_ref):
    a1 = jnp.zeros((BB, 1024), jnp.float32)
    for f in range(F):
        a1 = a1 + jnp.dot(emb_ref[f], w1_ref[f],
                          preferred_element_type=jnp.float32)
    h1 = jnp.maximum(a1 + b1_ref[...], 0.0)
    a2 = jnp.dot(h1, w2_ref[...], preferred_element_type=jnp.float32)
    h2 = jnp.maximum(a2 + b2_ref[...], 0.0)
    a3 = jnp.dot(h2, w3_ref[...], preferred_element_type=jnp.float32)
    h3 = jnp.maximum(a3 + b3_ref[...], 0.0)
    deep = jnp.sum(h3 * wout_ref[...], axis=1)
    out_ref[...] = deep + lin_ref[...] + cbias_ref[0, 0]


def _mlp(emb, lin, w1, b1, w2, b2, w3, b3, wout_t, cbias):
    grid = (B // BB,)
    return pl.pallas_call(
        _mlp_body,
        grid=grid,
        in_specs=[
            pl.BlockSpec((F, BB, D), lambda i: (0, i, 0)),
            pl.BlockSpec((BB,), lambda i: (i,)),
            pl.BlockSpec((F, D, 1024), lambda i: (0, 0, 0)),
            pl.BlockSpec((1, 1024), lambda i: (0, 0)),
            pl.BlockSpec((1024, 512), lambda i: (0, 0)),
            pl.BlockSpec((1, 512), lambda i: (0, 0)),
            pl.BlockSpec((512, 256), lambda i: (0, 0)),
            pl.BlockSpec((1, 256), lambda i: (0, 0)),
            pl.BlockSpec((1, 256), lambda i: (0, 0)),
            pl.BlockSpec((1, 1), lambda i: (0, 0)),
        ],
        out_specs=pl.BlockSpec((BB,), lambda i: (i,)),
        out_shape=jax.ShapeDtypeStruct((B,), jnp.float32),
    )(emb, lin, w1, b1, w2, b2, w3, b3, wout_t, cbias)


_OFFS = np.arange(F, dtype=np.int32) * FIELD


def kernel(x, linear_w, linear_bias, embed_table, W1, b1, W2, b2, W3, b3,
           Wout, bout):
    idx = x.astype(jnp.int32) + _OFFS[None, :]
    idx_fm = idx.T.reshape(NW, NCH, CH)
    idx_w = idx.reshape(NW, BW, F).transpose(0, 2, 1)
    emb_k, wide_k = _get_sc_kernels()
    emb = emb_k(idx_fm, embed_table)
    lin = wide_k(idx_w, linear_w.reshape(V))
    out = _mlp(
        emb.reshape(F, B, D), lin,
        W1.reshape(F, D, 1024), b1.reshape(1, 1024),
        W2, b2.reshape(1, 512),
        W3, b3.reshape(1, 256),
        Wout.reshape(1, 256), (bout + linear_bias).reshape(1, 1),
    )
    return out


# speedup vs baseline: 18.2037x; 1.2669x over previous
import functools

import jax
import jax.numpy as jnp
import numpy as np
from jax import lax
from jax.experimental import pallas as pl
from jax.experimental.pallas import tpu as pltpu
from jax.experimental.pallas import tpu_sc as plsc

B = 16384
F = 26
D = 128
FIELD = 4000
V = F * FIELD
EMB_OUT = F * D
B26 = B * F

NC = 2
NS = 16
NW = NC * NS
PER_W = B26 // NW
CH = 128
NCH = PER_W // CH
BW = B // NW


def _sc_emb_body(idx_hbm, tab_hbm, emb_out,
                 idx_v, bufA, bufB, semA, semB, semWA, semWB):
    wid = lax.axis_index("s") * NC + lax.axis_index("c")
    base = wid * PER_W
    pltpu.sync_copy(idx_hbm.at[wid], idx_v)

    def body(c, carry):
        c0 = c * 2
        c1 = c0 + 1

        @pl.when(c > 0)
        def _drain_prev_writes():
            pltpu.make_async_copy(bufA, emb_out.at[pl.ds(0, CH)], semWA).wait()
            pltpu.make_async_copy(bufB, emb_out.at[pl.ds(0, CH)], semWB).wait()

        gA = pltpu.async_copy(tab_hbm.at[idx_v.at[c0]], bufA, semA)
        gB = pltpu.async_copy(tab_hbm.at[idx_v.at[c1]], bufB, semB)
        gA.wait()
        pltpu.async_copy(bufA, emb_out.at[pl.ds(base + c0 * CH, CH)], semWA)
        gB.wait()
        pltpu.async_copy(bufB, emb_out.at[pl.ds(base + c1 * CH, CH)], semWB)
        return carry

    lax.fori_loop(0, NCH // 2, body, 0)
    pltpu.make_async_copy(bufA, emb_out.at[pl.ds(0, CH)], semWA).wait()
    pltpu.make_async_copy(bufB, emb_out.at[pl.ds(0, CH)], semWB).wait()


def _sc_wide_body(idx_hbm, lw_hbm, lin_out, idx_v, lw_v, acc_v, sem):
    wid = lax.axis_index("s") * NC + lax.axis_index("c")
    base = wid * BW
    cp_lw = pltpu.async_copy(lw_hbm, lw_v, sem)
    pltpu.sync_copy(idx_hbm.at[wid], idx_v)
    cp_lw.wait()

    def body(j, carry):
        p = j * 16
        acc = jnp.zeros((16,), jnp.float32)
        for f in range(F):
            ivec = idx_v[f, pl.ds(p, 16)]
            acc = acc + plsc.load_gather(lw_v, [ivec])
        acc_v[pl.ds(p, 16)] = acc
        return carry

    lax.fori_loop(0, BW // 16, body, 0)
    pltpu.sync_copy(acc_v, lin_out.at[pl.ds(base, BW)])


@functools.cache
def _get_sc_kernels():
    mesh = plsc.VectorSubcoreMesh(
        core_axis_name="c", subcore_axis_name="s",
        num_cores=NC, num_subcores=NS)
    emb_k = pl.kernel(
        _sc_emb_body,
        out_type=jax.ShapeDtypeStruct((B26, D), jnp.float32),
        mesh=mesh,
        scratch_types=[
            pltpu.VMEM((NCH, CH), jnp.int32),
            pltpu.VMEM((CH, D), jnp.float32),
            pltpu.VMEM((CH, D), jnp.float32),
            pltpu.SemaphoreType.DMA,
            pltpu.SemaphoreType.DMA,
            pltpu.SemaphoreType.DMA,
            pltpu.SemaphoreType.DMA,
        ],
    )
    wide_k = pl.kernel(
        _sc_wide_body,
        out_type=jax.ShapeDtypeStruct((B,), jnp.float32),
        mesh=mesh,
        scratch_types=[
            pltpu.VMEM((F, BW), jnp.int32),
            pltpu.VMEM((V,), jnp.float32),
            pltpu.VMEM((BW,), jnp.float32),
            pltpu.SemaphoreType.DMA,
        ],
        compiler_params=pltpu.CompilerParams(needs_layout_passes=False),
    )
    return emb_k, wide_k


BB = 512


def _mlp_body(emb_ref, lin_ref, w1_ref, b1_ref, w2_ref, b2_ref, w3_ref,
              b3_ref, wout_ref, cbias_ref, out_ref):
    a1 = jnp.zeros((BB, 1024), jnp.float32)
    for f in range(0, F, 2):
        lhs = jnp.concatenate(
            [emb_ref[f].astype(jnp.bfloat16),
             emb_ref[f + 1].astype(jnp.bfloat16)], axis=1)
        rhs = jnp.concatenate([w1_ref[f], w1_ref[f + 1]], axis=0)
        a1 = a1 + jnp.dot(lhs, rhs, preferred_element_type=jnp.float32)
    h1 = jnp.maximum(a1 + b1_ref[...], 0.0).astype(jnp.bfloat16)
    a2 = jnp.dot(h1, w2_ref[...], preferred_element_type=jnp.float32)
    h2 = jnp.maximum(a2 + b2_ref[...], 0.0).astype(jnp.bfloat16)
    a3 = jnp.dot(h2, w3_ref[...], preferred_element_type=jnp.float32)
    h3 = jnp.maximum(a3 + b3_ref[...], 0.0)
    deep = jnp.sum(h3 * wout_ref[...], axis=1)
    out_ref[...] = deep + lin_ref[...] + cbias_ref[0, 0]


def _mlp(emb, lin, w1, b1, w2, b2, w3, b3, wout_t, cbias):
    grid = (B // BB,)
    return pl.pallas_call(
        _mlp_body,
        grid=grid,
        in_specs=[
            pl.BlockSpec((F, BB, D), lambda i: (0, i, 0)),
            pl.BlockSpec((BB,), lambda i: (i,)),
            pl.BlockSpec((F, D, 1024), lambda i: (0, 0, 0)),
            pl.BlockSpec((1, 1024), lambda i: (0, 0)),
            pl.BlockSpec((1024, 512), lambda i: (0, 0)),
            pl.BlockSpec((1, 512), lambda i: (0, 0)),
            pl.BlockSpec((512, 256), lambda i: (0, 0)),
            pl.BlockSpec((1, 256), lambda i: (0, 0)),
            pl.BlockSpec((1, 256), lambda i: (0, 0)),
            pl.BlockSpec((1, 1), lambda i: (0, 0)),
        ],
        out_specs=pl.BlockSpec((BB,), lambda i: (i,)),
        out_shape=jax.ShapeDtypeStruct((B,), jnp.float32),
    )(emb, lin, w1, b1, w2, b2, w3, b3, wout_t, cbias)


_OFFS = np.arange(F, dtype=np.int32) * FIELD


def kernel(x, linear_w, linear_bias, embed_table, W1, b1, W2, b2, W3, b3,
           Wout, bout):
    idx = x.astype(jnp.int32) + _OFFS[None, :]
    idx_fm = idx.T.reshape(NW, NCH, CH)
    idx_w = idx.reshape(NW, BW, F).transpose(0, 2, 1)
    emb_k, wide_k = _get_sc_kernels()
    emb = emb_k(idx_fm, embed_table)
    lin = wide_k(idx_w, linear_w.reshape(V))
    out = _mlp(
        emb.reshape(F, B, D), lin,
        W1.reshape(F, D, 1024).astype(jnp.bfloat16), b1.reshape(1, 1024),
        W2.astype(jnp.bfloat16), b2.reshape(1, 512),
        W3.astype(jnp.bfloat16), b3.reshape(1, 256),
        Wout.reshape(1, 256), (bout + linear_bias).reshape(1, 1),
    )
    return out


# speedup vs baseline: 22.0383x; 1.2107x over previous
import functools

import jax
import jax.numpy as jnp
import numpy as np
from jax import lax
from jax.experimental import pallas as pl
from jax.experimental.pallas import tpu as pltpu
from jax.experimental.pallas import tpu_sc as plsc

B = 16384
F = 26
D = 128
FIELD = 4000
V = F * FIELD
NCK = 4
CB = B // NCK

NC = 2
NS = 16
NW = NC * NS
CH = 128
PER_W = F * CB // NW
NCH = PER_W // CH
BW = B // NW


def _sc_emb_body(idx_hbm, tab_hbm, emb_out,
                 idx_v, bufA, bufB, semA, semB, semWA, semWB):
    wid = lax.axis_index("s") * NC + lax.axis_index("c")
    base = wid * PER_W
    pltpu.sync_copy(idx_hbm.at[wid], idx_v)

    def body(c, carry):
        c0 = c * 2
        c1 = c0 + 1

        @pl.when(c > 0)
        def _drain_prev_writes():
            pltpu.make_async_copy(bufA, emb_out.at[pl.ds(0, CH)], semWA).wait()
            pltpu.make_async_copy(bufB, emb_out.at[pl.ds(0, CH)], semWB).wait()

        gA = pltpu.async_copy(tab_hbm.at[idx_v.at[c0]], bufA, semA)
        gB = pltpu.async_copy(tab_hbm.at[idx_v.at[c1]], bufB, semB)
        gA.wait()
        pltpu.async_copy(bufA, emb_out.at[pl.ds(base + c0 * CH, CH)], semWA)
        gB.wait()
        pltpu.async_copy(bufB, emb_out.at[pl.ds(base + c1 * CH, CH)], semWB)
        return carry

    lax.fori_loop(0, NCH // 2, body, 0)
    pltpu.make_async_copy(bufA, emb_out.at[pl.ds(0, CH)], semWA).wait()
    pltpu.make_async_copy(bufB, emb_out.at[pl.ds(0, CH)], semWB).wait()


def _sc_wide_body(idx_hbm, lw_hbm, lin_out, idx_v, lw_v, acc_v, sem):
    wid = lax.axis_index("s") * NC + lax.axis_index("c")
    base = wid * BW
    cp_lw = pltpu.async_copy(lw_hbm, lw_v, sem)
    pltpu.sync_copy(idx_hbm.at[wid], idx_v)
    cp_lw.wait()

    def body(j, carry):
        p = j * 16
        acc = jnp.zeros((16,), jnp.float32)
        for f in range(F):
            ivec = idx_v[f, pl.ds(p, 16)]
            acc = acc + plsc.load_gather(lw_v, [ivec])
        acc_v[pl.ds(p, 16)] = acc
        return carry

    lax.fori_loop(0, BW // 16, body, 0)
    pltpu.sync_copy(acc_v, lin_out.at[pl.ds(base, BW)])


@functools.cache
def _get_sc_kernels():
    mesh = plsc.VectorSubcoreMesh(
        core_axis_name="c", subcore_axis_name="s",
        num_cores=NC, num_subcores=NS)
    emb_k = pl.kernel(
        _sc_emb_body,
        out_type=jax.ShapeDtypeStruct((F * CB, D), jnp.float32),
        mesh=mesh,
        scratch_types=[
            pltpu.VMEM((NCH, CH), jnp.int32),
            pltpu.VMEM((CH, D), jnp.float32),
            pltpu.VMEM((CH, D), jnp.float32),
            pltpu.SemaphoreType.DMA,
            pltpu.SemaphoreType.DMA,
            pltpu.SemaphoreType.DMA,
            pltpu.SemaphoreType.DMA,
        ],
    )
    wide_k = pl.kernel(
        _sc_wide_body,
        out_type=jax.ShapeDtypeStruct((B,), jnp.float32),
        mesh=mesh,
        scratch_types=[
            pltpu.VMEM((F, BW), jnp.int32),
            pltpu.VMEM((V,), jnp.float32),
            pltpu.VMEM((BW,), jnp.float32),
            pltpu.SemaphoreType.DMA,
        ],
        compiler_params=pltpu.CompilerParams(needs_layout_passes=False),
    )
    return emb_k, wide_k


BB = 512


def _mlp_body(emb_ref, lin_ref, w1_ref, b1_ref, w2_ref, b2_ref, w3_ref,
              b3_ref, wout_ref, cbias_ref, out_ref):
    a1 = jnp.zeros((BB, 1024), jnp.float32)
    for f in range(0, F, 2):
        lhs = jnp.concatenate(
            [emb_ref[f].astype(jnp.bfloat16),
             emb_ref[f + 1].astype(jnp.bfloat16)], axis=1)
        rhs = jnp.concatenate([w1_ref[f], w1_ref[f + 1]], axis=0)
        a1 = a1 + jnp.dot(lhs, rhs, preferred_element_type=jnp.float32)
    h1 = jnp.maximum(a1 + b1_ref[...], 0.0).astype(jnp.bfloat16)
    a2 = jnp.dot(h1, w2_ref[...], preferred_element_type=jnp.float32)
    h2 = jnp.maximum(a2 + b2_ref[...], 0.0).astype(jnp.bfloat16)
    a3 = jnp.dot(h2, w3_ref[...], preferred_element_type=jnp.float32)
    h3 = jnp.maximum(a3 + b3_ref[...], 0.0)
    deep = jnp.sum(h3 * wout_ref[...], axis=1)
    out_ref[...] = deep + lin_ref[...] + cbias_ref[0, 0]


def _mlp(emb, lin, w1, b1, w2, b2, w3, b3, wout_t, cbias):
    grid = (CB // BB,)
    return pl.pallas_call(
        _mlp_body,
        grid=grid,
        in_specs=[
            pl.BlockSpec((F, BB, D), lambda i: (0, i, 0)),
            pl.BlockSpec((BB,), lambda i: (i,)),
            pl.BlockSpec((F, D, 1024), lambda i: (0, 0, 0)),
            pl.BlockSpec((1, 1024), lambda i: (0, 0)),
            pl.BlockSpec((1024, 512), lambda i: (0, 0)),
            pl.BlockSpec((1, 512), lambda i: (0, 0)),
            pl.BlockSpec((512, 256), lambda i: (0, 0)),
            pl.BlockSpec((1, 256), lambda i: (0, 0)),
            pl.BlockSpec((1, 256), lambda i: (0, 0)),
            pl.BlockSpec((1, 1), lambda i: (0, 0)),
        ],
        out_specs=pl.BlockSpec((BB,), lambda i: (i,)),
        out_shape=jax.ShapeDtypeStruct((CB,), jnp.float32),
    )(emb, lin, w1, b1, w2, b2, w3, b3, wout_t, cbias)


_OFFS = np.arange(F, dtype=np.int32) * FIELD


def kernel(x, linear_w, linear_bias, embed_table, W1, b1, W2, b2, W3, b3,
           Wout, bout):
    idx = x.astype(jnp.int32) + _OFFS[None, :]
    idx_w = idx.reshape(NW, BW, F).transpose(0, 2, 1)
    emb_k, wide_k = _get_sc_kernels()
    lin = wide_k(idx_w, linear_w.reshape(V))

    w1r = W1.reshape(F, D, 1024).astype(jnp.bfloat16)
    w2c = W2.astype(jnp.bfloat16)
    w3c = W3.astype(jnp.bfloat16)
    b1r = b1.reshape(1, 1024)
    b2r = b2.reshape(1, 512)
    b3r = b3.reshape(1, 256)
    woutr = Wout.reshape(1, 256)
    cbias = (bout + linear_bias).reshape(1, 1)

    embs = []
    for c in range(NCK):
        idx_c = idx[c * CB:(c + 1) * CB].T.reshape(NW, NCH, CH)
        embs.append(emb_k(idx_c, embed_table))
    outs = []
    for c in range(NCK):
        lin_c = lax.slice(lin, (c * CB,), ((c + 1) * CB,))
        outs.append(_mlp(embs[c].reshape(F, CB, D), lin_c,
                         w1r, b1r, w2c, b2r, w3c, b3r, woutr, cbias))
    return jnp.concatenate(outs)


# speedup vs baseline: 23.2708x; 1.0559x over previous
import functools

import jax
import jax.numpy as jnp
import numpy as np
from jax import lax
from jax.experimental import pallas as pl
from jax.experimental.pallas import tpu as pltpu
from jax.experimental.pallas import tpu_sc as plsc

B = 16384
F = 26
D = 128
FIELD = 4000
V = F * FIELD
NCK = 4
CB = B // NCK

NC = 2
NS = 16
NW = NC * NS
CH = 128
PER_W = F * CB // NW
NCH = PER_W // CH
BW = B // NW


def _sc_emb_body(idx_hbm, tab_hbm, emb_out,
                 idx_v, bufA, bufB, semA, semB, semWA, semWB):
    wid = lax.axis_index("s") * NC + lax.axis_index("c")
    base = wid * PER_W
    pltpu.sync_copy(idx_hbm.at[wid], idx_v)

    def body(c, carry):
        c0 = c * 2
        c1 = c0 + 1

        @pl.when(c > 0)
        def _drain_prev_writes():
            pltpu.make_async_copy(bufA, emb_out.at[pl.ds(0, CH)], semWA).wait()
            pltpu.make_async_copy(bufB, emb_out.at[pl.ds(0, CH)], semWB).wait()

        gA = pltpu.async_copy(tab_hbm.at[idx_v.at[c0]], bufA, semA)
        gB = pltpu.async_copy(tab_hbm.at[idx_v.at[c1]], bufB, semB)
        gA.wait()
        pltpu.async_copy(bufA, emb_out.at[pl.ds(base + c0 * CH, CH)], semWA)
        gB.wait()
        pltpu.async_copy(bufB, emb_out.at[pl.ds(base + c1 * CH, CH)], semWB)
        return carry

    lax.fori_loop(0, NCH // 2, body, 0)
    pltpu.make_async_copy(bufA, emb_out.at[pl.ds(0, CH)], semWA).wait()
    pltpu.make_async_copy(bufB, emb_out.at[pl.ds(0, CH)], semWB).wait()


def _sc_wide_body(idx_hbm, lw_hbm, lin_out, idx_v, lw_v, acc_v, sem):
    wid = lax.axis_index("s") * NC + lax.axis_index("c")
    base = wid * BW
    cp_lw = pltpu.async_copy(lw_hbm, lw_v, sem)
    pltpu.sync_copy(idx_hbm.at[wid], idx_v)
    cp_lw.wait()

    def body(j, carry):
        p = j * 16
        acc = jnp.zeros((16,), jnp.float32)
        for f in range(F):
            ivec = idx_v[f, pl.ds(p, 16)]
            acc = acc + plsc.load_gather(lw_v, [ivec])
        acc_v[pl.ds(p, 16)] = acc
        return carry

    lax.fori_loop(0, BW // 16, body, 0)
    pltpu.sync_copy(acc_v, lin_out.at[pl.ds(base, BW)])


@functools.cache
def _get_sc_kernels():
    mesh = plsc.VectorSubcoreMesh(
        core_axis_name="c", subcore_axis_name="s",
        num_cores=NC, num_subcores=NS)
    emb_k = pl.kernel(
        _sc_emb_body,
        out_type=jax.ShapeDtypeStruct((F * CB, D), jnp.float32),
        mesh=mesh,
        scratch_types=[
            pltpu.VMEM((NCH, CH), jnp.int32),
            pltpu.VMEM((CH, D), jnp.float32),
            pltpu.VMEM((CH, D), jnp.float32),
            pltpu.SemaphoreType.DMA,
            pltpu.SemaphoreType.DMA,
            pltpu.SemaphoreType.DMA,
            pltpu.SemaphoreType.DMA,
        ],
    )
    wide_k = pl.kernel(
        _sc_wide_body,
        out_type=jax.ShapeDtypeStruct((B,), jnp.float32),
        mesh=mesh,
        scratch_types=[
            pltpu.VMEM((F, BW), jnp.int32),
            pltpu.VMEM((V,), jnp.float32),
            pltpu.VMEM((BW,), jnp.float32),
            pltpu.SemaphoreType.DMA,
        ],
        compiler_params=pltpu.CompilerParams(needs_layout_passes=False),
    )
    return emb_k, wide_k


BB = 512


def _mlp_body(emb_ref, w1_ref, b1_ref, w2_ref, b2_ref, w3_ref,
              b3_ref, wout_ref, out_ref):
    a1 = jnp.zeros((BB, 1024), jnp.float32)
    for f in range(0, F, 2):
        lhs = jnp.concatenate(
            [emb_ref[f].astype(jnp.bfloat16),
             emb_ref[f + 1].astype(jnp.bfloat16)], axis=1)
        rhs = jnp.concatenate([w1_ref[f], w1_ref[f + 1]], axis=0)
        a1 = a1 + jnp.dot(lhs, rhs, preferred_element_type=jnp.float32)
    h1 = jnp.maximum(a1 + b1_ref[...], 0.0).astype(jnp.bfloat16)
    a2 = jnp.dot(h1, w2_ref[...], preferred_element_type=jnp.float32)
    h2 = jnp.maximum(a2 + b2_ref[...], 0.0).astype(jnp.bfloat16)
    a3 = jnp.dot(h2, w3_ref[...], preferred_element_type=jnp.float32)
    h3 = jnp.maximum(a3 + b3_ref[...], 0.0)
    deep = jnp.sum(h3 * wout_ref[...], axis=1)
    out_ref[...] = deep


def _mlp(emb, w1, b1, w2, b2, w3, b3, wout_t):
    grid = (CB // BB,)
    return pl.pallas_call(
        _mlp_body,
        grid=grid,
        in_specs=[
            pl.BlockSpec((F, BB, D), lambda i: (0, i, 0)),
            pl.BlockSpec((F, D, 1024), lambda i: (0, 0, 0)),
            pl.BlockSpec((1, 1024), lambda i: (0, 0)),
            pl.BlockSpec((1024, 512), lambda i: (0, 0)),
            pl.BlockSpec((1, 512), lambda i: (0, 0)),
            pl.BlockSpec((512, 256), lambda i: (0, 0)),
            pl.BlockSpec((1, 256), lambda i: (0, 0)),
            pl.BlockSpec((1, 256), lambda i: (0, 0)),
        ],
        out_specs=pl.BlockSpec((BB,), lambda i: (i,)),
        out_shape=jax.ShapeDtypeStruct((CB,), jnp.float32),
    )(emb, w1, b1, w2, b2, w3, b3, wout_t)


_OFFS = np.arange(F, dtype=np.int32) * FIELD


def kernel(x, linear_w, linear_bias, embed_table, W1, b1, W2, b2, W3, b3,
           Wout, bout):
    idx = x.astype(jnp.int32) + _OFFS[None, :]
    idx_w = idx.reshape(NW, BW, F).transpose(0, 2, 1)
    emb_k, wide_k = _get_sc_kernels()

    embs = []
    for c in range(NCK):
        idx_c = idx[c * CB:(c + 1) * CB].T.reshape(NW, NCH, CH)
        embs.append(emb_k(idx_c, embed_table))
    lin = wide_k(idx_w, linear_w.reshape(V))

    w1r = W1.reshape(F, D, 1024).astype(jnp.bfloat16)
    w2c = W2.astype(jnp.bfloat16)
    w3c = W3.astype(jnp.bfloat16)
    b1r = b1.reshape(1, 1024)
    b2r = b2.reshape(1, 512)
    b3r = b3.reshape(1, 256)
    woutr = Wout.reshape(1, 256)

    outs = []
    for c in range(NCK):
        outs.append(_mlp(embs[c].reshape(F, CB, D),
                         w1r, b1r, w2c, b2r, w3c, b3r, woutr))
    deep = jnp.concatenate(outs)
    return deep + lin + (bout[0] + linear_bias[0])


# speedup vs baseline: 23.2885x; 1.0008x over previous
import functools

import jax
import jax.numpy as jnp
import numpy as np
from jax import lax
from jax.experimental import pallas as pl
from jax.experimental.pallas import tpu as pltpu
from jax.experimental.pallas import tpu_sc as plsc

B = 16384
F = 26
D = 128
FIELD = 4000
V = F * FIELD
NCK = 4
CB = B // NCK

NC = 2
NS = 16
NW = NC * NS
CH = 128
PER_W = F * CB // NW
NCH = PER_W // CH
BW = B // NW


def _sc_emb_body(idx_hbm, tab_hbm, emb_out,
                 idx_v, bufA, bufB, semA, semB, semWA, semWB):
    wid = lax.axis_index("s") * NC + lax.axis_index("c")
    base = wid * PER_W
    pltpu.sync_copy(idx_hbm.at[wid], idx_v)

    def body(c, carry):
        c0 = c * 2
        c1 = c0 + 1

        @pl.when(c > 0)
        def _drain_prev_writes():
            pltpu.make_async_copy(bufA, emb_out.at[pl.ds(0, CH)], semWA).wait()
            pltpu.make_async_copy(bufB, emb_out.at[pl.ds(0, CH)], semWB).wait()

        gA = pltpu.async_copy(tab_hbm.at[idx_v.at[c0]], bufA, semA)
        gB = pltpu.async_copy(tab_hbm.at[idx_v.at[c1]], bufB, semB)
        gA.wait()
        pltpu.async_copy(bufA, emb_out.at[pl.ds(base + c0 * CH, CH)], semWA)
        gB.wait()
        pltpu.async_copy(bufB, emb_out.at[pl.ds(base + c1 * CH, CH)], semWB)
        return carry

    lax.fori_loop(0, NCH // 2, body, 0)
    pltpu.make_async_copy(bufA, emb_out.at[pl.ds(0, CH)], semWA).wait()
    pltpu.make_async_copy(bufB, emb_out.at[pl.ds(0, CH)], semWB).wait()


def _sc_wide_body(idx_hbm, lw_hbm, lin_out, idx_v, lw_v, acc_v, sem):
    wid = lax.axis_index("s") * NC + lax.axis_index("c")
    base = wid * BW
    cp_lw = pltpu.async_copy(lw_hbm, lw_v, sem)
    pltpu.sync_copy(idx_hbm.at[wid], idx_v)
    cp_lw.wait()

    def body(j, carry):
        p = j * 16
        acc = jnp.zeros((16,), jnp.float32)
        for f in range(F):
            ivec = idx_v[f, pl.ds(p, 16)]
            acc = acc + plsc.load_gather(lw_v, [ivec])
        acc_v[pl.ds(p, 16)] = acc
        return carry

    lax.fori_loop(0, BW // 16, body, 0)
    pltpu.sync_copy(acc_v, lin_out.at[pl.ds(base, BW)])


@functools.cache
def _get_sc_kernels():
    mesh = plsc.VectorSubcoreMesh(
        core_axis_name="c", subcore_axis_name="s",
        num_cores=NC, num_subcores=NS)
    emb_k = pl.kernel(
        _sc_emb_body,
        out_type=jax.ShapeDtypeStruct((F * CB, D), jnp.float32),
        mesh=mesh,
        scratch_types=[
            pltpu.VMEM((NCH, CH), jnp.int32),
            pltpu.VMEM((CH, D), jnp.float32),
            pltpu.VMEM((CH, D), jnp.float32),
            pltpu.SemaphoreType.DMA,
            pltpu.SemaphoreType.DMA,
            pltpu.SemaphoreType.DMA,
            pltpu.SemaphoreType.DMA,
        ],
        compiler_params=pltpu.CompilerParams(has_side_effects=True),
    )
    wide_k = pl.kernel(
        _sc_wide_body,
        out_type=jax.ShapeDtypeStruct((B,), jnp.float32),
        mesh=mesh,
        scratch_types=[
            pltpu.VMEM((F, BW), jnp.int32),
            pltpu.VMEM((V,), jnp.float32),
            pltpu.VMEM((BW,), jnp.float32),
            pltpu.SemaphoreType.DMA,
        ],
        compiler_params=pltpu.CompilerParams(
            needs_layout_passes=False, has_side_effects=True),
    )
    return emb_k, wide_k


BB = 512


def _mlp_body(emb_ref, w1_ref, b1_ref, w2_ref, b2_ref, w3_ref,
              b3_ref, wout_ref, out_ref):
    a1 = jnp.zeros((BB, 1024), jnp.float32)
    for f in range(0, F, 2):
        lhs = jnp.concatenate(
            [emb_ref[f].astype(jnp.bfloat16),
             emb_ref[f + 1].astype(jnp.bfloat16)], axis=1)
        rhs = jnp.concatenate([w1_ref[f], w1_ref[f + 1]], axis=0)
        a1 = a1 + jnp.dot(lhs, rhs, preferred_element_type=jnp.float32)
    h1 = jnp.maximum(a1 + b1_ref[...], 0.0).astype(jnp.bfloat16)
    a2 = jnp.dot(h1, w2_ref[...], preferred_element_type=jnp.float32)
    h2 = jnp.maximum(a2 + b2_ref[...], 0.0).astype(jnp.bfloat16)
    a3 = jnp.dot(h2, w3_ref[...], preferred_element_type=jnp.float32)
    h3 = jnp.maximum(a3 + b3_ref[...], 0.0)
    deep = jnp.sum(h3 * wout_ref[...], axis=1)
    out_ref[...] = deep


def _mlp(emb, w1, b1, w2, b2, w3, b3, wout_t):
    grid = (CB // BB,)
    return pl.pallas_call(
        _mlp_body,
        grid=grid,
        in_specs=[
            pl.BlockSpec((F, BB, D), lambda i: (0, i, 0)),
            pl.BlockSpec((F, D, 1024), lambda i: (0, 0, 0)),
            pl.BlockSpec((1, 1024), lambda i: (0, 0)),
            pl.BlockSpec((1024, 512), lambda i: (0, 0)),
            pl.BlockSpec((1, 512), lambda i: (0, 0)),
            pl.BlockSpec((512, 256), lambda i: (0, 0)),
            pl.BlockSpec((1, 256), lambda i: (0, 0)),
            pl.BlockSpec((1, 256), lambda i: (0, 0)),
        ],
        out_specs=pl.BlockSpec((BB,), lambda i: (i,)),
        out_shape=jax.ShapeDtypeStruct((CB,), jnp.float32),
    )(emb, w1, b1, w2, b2, w3, b3, wout_t)


_OFFS = np.arange(F, dtype=np.int32) * FIELD


def kernel(x, linear_w, linear_bias, embed_table, W1, b1, W2, b2, W3, b3,
           Wout, bout):
    idx = x.astype(jnp.int32) + _OFFS[None, :]
    idx_w = idx.reshape(NW, BW, F).transpose(0, 2, 1)
    emb_k, wide_k = _get_sc_kernels()

    embs = []
    for c in range(NCK):
        idx_c = idx[c * CB:(c + 1) * CB].T.reshape(NW, NCH, CH)
        embs.append(emb_k(idx_c, embed_table))
    lin = wide_k(idx_w, linear_w.reshape(V))

    w1r = W1.reshape(F, D, 1024).astype(jnp.bfloat16)
    w2c = W2.astype(jnp.bfloat16)
    w3c = W3.astype(jnp.bfloat16)
    b1r = b1.reshape(1, 1024)
    b2r = b2.reshape(1, 512)
    b3r = b3.reshape(1, 256)
    woutr = Wout.reshape(1, 256)

    outs = []
    for c in range(NCK):
        outs.append(_mlp(embs[c].reshape(F, CB, D),
                         w1r, b1r, w2c, b2r, w3c, b3r, woutr))
    deep = jnp.concatenate(outs)
    return deep + lin + (bout[0] + linear_bias[0])
